# baseline (speedup 1.0000x reference)
import functools

import jax
import jax.numpy as jnp
from jax import lax
from jax.experimental import pallas as pl
from jax.experimental.pallas import tpu as pltpu
from jax.experimental.pallas import tpu_sc as plsc

NC, NS = 2, 16
NW = NC * NS

B, L, V, D = 4096, 200, 1000000, 64
F = 4
BB = B // NW
NFIX = BB * F
NBUF = 4
TBUF = 2
MFIX = 32
NGRP = MFIX // 16


def _sc_body(ids_hbm, table_hbm, slots_hbm, cnts_hbm, words_hbm, we_hbm,
             out_hbm, idx_v, rows_v, trows_v, fvecs_v, fwords_v, slots_v,
             cnts_v, gsem, ssem, fgsem):
    c = lax.axis_index("c")
    s = lax.axis_index("s")
    w = s * NC + c

    pltpu.sync_copy(ids_hbm.at[w], idx_v)
    pltpu.sync_copy(words_hbm.at[w], fwords_v)
    pltpu.sync_copy(slots_hbm.at[w], slots_v)
    pltpu.sync_copy(cnts_hbm.at[w], cnts_v)
    for j in range(F):
        pltpu.async_copy(we_hbm.at[fwords_v.at[j]],
                         fvecs_v.at[pl.ds(j * BB, BB)], fgsem)
    for j in range(F):
        pltpu.make_async_copy(we_hbm.at[fwords_v.at[0]],
                              fvecs_v.at[pl.ds(0, BB)], fgsem).wait()

    lane = lax.iota(jnp.int32, 16)
    qvs, dvs = [], []
    for t in range(4):
        dfull = lane + 16 * t
        qvs.append(lax.shift_right_logical(dfull, 3))
        dvs.append(lax.bitwise_and(dfull, 7))

    def fire_gather(l, bslot):
        pltpu.async_copy(table_hbm.at[idx_v.at[l]], rows_v.at[bslot],
                         gsem.at[bslot])

    def wait_gather(bslot):
        pltpu.make_async_copy(table_hbm.at[idx_v.at[0]], rows_v.at[bslot],
                              gsem.at[bslot]).wait()

    def fire_store(l, tslot):
        pltpu.async_copy(trows_v.at[tslot], out_hbm.at[l, :, w],
                         ssem.at[tslot])

    def wait_store(tslot):
        pltpu.make_async_copy(trows_v.at[tslot], out_hbm.at[0, :, w],
                              ssem.at[tslot]).wait()

    for bslot in range(NBUF):
        fire_gather(bslot, bslot)

    @pl.loop(0, L, step=NBUF)
    def _group(g0):
        for bslot in range(NBUF):
            l = g0 + bslot
            tslot = bslot % TBUF
            wait_gather(bslot)

            cnt = cnts_v[l, pl.ds(0, 16)][0]
            for g in range(NGRP):
                @pl.when(cnt > g * 16)
                def _fix_group():
                    sv = slots_v[l, pl.ds(g * 16, 16)]
                    for i in range(16):
                        @pl.when(g * 16 + i < cnt)
                        def _one_fix():
                            slot = sv[i]
                            col = lax.shift_right_logical(slot, 2)
                            for t in range(4):
                                rows_v[bslot, col, pl.ds(t * 16, 16)] = (
                                    fvecs_v[slot, pl.ds(t * 16, 16)])

            @pl.when(l >= TBUF)
            def _free():
                wait_store(tslot)

            @plsc.parallel_loop(0, BB, 1, unroll=8)
            def _col(b_lo):
                bv = jnp.full((16,), b_lo, jnp.int32)
                for t in range(4):
                    vals = rows_v[bslot, b_lo, pl.ds(t * 16, 16)]
                    plsc.store_scatter(trows_v.at[tslot], [qvs[t], dvs[t], bv],
                                       vals)

            fire_store(l, tslot)

            @pl.when(l + NBUF < L)
            def _refill():
                fire_gather(l + NBUF, bslot)

    for tslot in range(TBUF):
        wait_store(tslot)


@jax.jit
def _embed_with_fixes(idsT3, table, slots3, cnts3, words3, word_embeddings):
    mesh = plsc.VectorSubcoreMesh(
        core_axis_name="c", subcore_axis_name="s",
        num_cores=NC, num_subcores=NS)
    return pl.kernel(
        _sc_body,
        out_type=jax.ShapeDtypeStruct((L, D // 8, NW, 8, 128), jnp.float32),
        mesh=mesh,
        compiler_params=pltpu.CompilerParams(
            use_tc_tiling_on_sc=False, needs_layout_passes=False),
        scratch_types=[
            pltpu.VMEM((L, 128), jnp.int32),
            pltpu.VMEM((NBUF, BB, D), jnp.float32),
            pltpu.VMEM((TBUF, 8, 8, 128), jnp.float32),
            pltpu.VMEM((NFIX, D), jnp.float32),
            pltpu.VMEM((F, 128), jnp.int32),
            pltpu.VMEM((L, MFIX), jnp.int32),
            pltpu.VMEM((L, 16), jnp.int32),
            pltpu.SemaphoreType.DMA((NBUF,)),
            pltpu.SemaphoreType.DMA((TBUF,)),
            pltpu.SemaphoreType.DMA,
        ],
    )(idsT3, table, slots3, cnts3, words3, word_embeddings)


def kernel(input_ids, fix_offsets, fix_words, table, word_embeddings):
    idsT3 = input_ids.T.reshape(L, NW, 128).transpose(1, 0, 2)

    f_ids = jnp.arange(F, dtype=jnp.int32)
    eq = fix_offsets[:, :, None] == fix_offsets[:, None, :]
    last = jnp.max(jnp.where(eq, f_ids[None, None, :], -1), axis=2)
    win_words = jnp.take_along_axis(fix_words, last, axis=1)
    words3 = win_words.reshape(NW, F, 128)

    l_arr = fix_offsets.reshape(NW, NFIX)
    slot_ids = jnp.arange(NFIX, dtype=jnp.int32)[None, :].repeat(NW, axis=0)
    order = jnp.argsort(l_arr, axis=1, stable=True)
    sorted_l = jnp.take_along_axis(l_arr, order, axis=1)
    sorted_slots = jnp.take_along_axis(slot_ids, order, axis=1)
    grid_l = jnp.arange(L, dtype=jnp.int32)
    starts = jax.vmap(lambda a: jnp.searchsorted(a, grid_l, side="left"))(
        sorted_l).astype(jnp.int32)
    ends = jax.vmap(lambda a: jnp.searchsorted(a, grid_l, side="right"))(
        sorted_l).astype(jnp.int32)
    cnts = ends - starts
    pos = jnp.arange(NFIX, dtype=jnp.int32)[None, :] - jnp.take_along_axis(
        starts, sorted_l, axis=1)

    def _fill(sl, sp, ss):
        return jnp.zeros((L, MFIX), jnp.int32).at[sl, sp].set(ss, mode="drop")

    slots3 = jax.vmap(_fill)(sorted_l, pos, sorted_slots)
    cnts3 = jnp.minimum(cnts, MFIX)[:, :, None].repeat(16, axis=2)

    out = _embed_with_fixes(idsT3, table, slots3, cnts3, words3,
                            word_embeddings)
    return out.transpose(2, 4, 0, 1, 3).reshape(B, L, D)


# speedup vs baseline: 1.4521x; 1.4521x over previous
import functools

import jax
import jax.numpy as jnp
from jax import lax
from jax.experimental import pallas as pl
from jax.experimental.pallas import tpu as pltpu
from jax.experimental.pallas import tpu_sc as plsc

NC, NS = 2, 16
NW = NC * NS

B, L, V, D = 4096, 200, 1000000, 64
F = 4
BB = B // NW
NFIX = BB * F
HALF = L // 2
NBUF = 4


def _sc_body(ids_hbm, table_hbm, offs_hbm, words_hbm, we_hbm, out_hbm,
             idx_v, rows_v, fvecs_v, fwords_v, loffs_v, gsem, ssem, fsem):
    c = lax.axis_index("c")
    s = lax.axis_index("s")
    w = s * NC + c

    pltpu.sync_copy(ids_hbm.at[w], idx_v)
    pltpu.sync_copy(words_hbm.at[w], fwords_v)
    pltpu.sync_copy(offs_hbm.at[w], loffs_v)
    for j in range(F):
        pltpu.async_copy(we_hbm.at[fwords_v.at[j]],
                         fvecs_v.at[pl.ds(j * BB, BB)], fsem)
    for j in range(F):
        pltpu.make_async_copy(we_hbm.at[fwords_v.at[0]],
                              fvecs_v.at[pl.ds(0, BB)], fsem).wait()

    def fire_gather(bl, slot):
        for j in range(2):
            pltpu.async_copy(table_hbm.at[idx_v.at[bl, j]],
                             rows_v.at[slot, pl.ds(j * HALF, HALF)],
                             gsem.at[slot])

    def wait_gather(slot):
        for j in range(2):
            pltpu.make_async_copy(table_hbm.at[idx_v.at[0, 0]],
                                  rows_v.at[slot, pl.ds(0, HALF)],
                                  gsem.at[slot]).wait()

    def fire_store(bl, slot):
        pltpu.async_copy(rows_v.at[slot], out_hbm.at[w * BB + bl],
                         ssem.at[slot])

    def wait_store(slot):
        pltpu.make_async_copy(rows_v.at[slot], out_hbm.at[0], ssem.at[slot]
                              ).wait()

    for slot in range(NBUF):
        fire_gather(slot, slot)

    @pl.loop(0, BB, step=NBUF)
    def _group(g0):
        for slot in range(NBUF):
            bl = g0 + slot
            wait_gather(slot)
            fire_store(bl, slot)

            @pl.when(bl + NBUF < BB)
            def _refill():
                wait_store(slot)
                fire_gather(bl + NBUF, slot)

    for slot in range(NBUF):
        wait_store(slot)

    @pl.loop(0, NFIX // 16)
    def _fix_group(k16):
        lv = loffs_v[pl.ds(k16 * 16, 16)]
        for i in range(16):
            k = k16 * 16 + i
            l_k = lv[i]
            pltpu.async_copy(fvecs_v.at[k],
                             out_hbm.at[w * BB + k // F, l_k], fsem)
        for i in range(16):
            pltpu.make_async_copy(fvecs_v.at[0], out_hbm.at[0, 0], fsem
                                  ).wait()


@jax.jit
def _embed_with_fixes(ids4, table, offs2, words3, word_embeddings):
    mesh = plsc.VectorSubcoreMesh(
        core_axis_name="c", subcore_axis_name="s",
        num_cores=NC, num_subcores=NS)
    return pl.kernel(
        _sc_body,
        out_type=jax.ShapeDtypeStruct((B, L, D), jnp.float32),
        mesh=mesh,
        compiler_params=pltpu.CompilerParams(
            use_tc_tiling_on_sc=False, needs_layout_passes=False),
        scratch_types=[
            pltpu.VMEM((BB, 2, HALF), jnp.int32),
            pltpu.VMEM((NBUF, L, D), jnp.float32),
            pltpu.VMEM((NFIX, D), jnp.float32),
            pltpu.VMEM((F, 128), jnp.int32),
            pltpu.VMEM((NFIX,), jnp.int32),
            pltpu.SemaphoreType.DMA((NBUF,)),
            pltpu.SemaphoreType.DMA((NBUF,)),
            pltpu.SemaphoreType.DMA,
        ],
    )(ids4, table, offs2, words3, word_embeddings)


def kernel(input_ids, fix_offsets, fix_words, table, word_embeddings):
    ids4 = input_ids.reshape(NW, BB, 2, HALF)

    f_ids = jnp.arange(F, dtype=jnp.int32)
    eq = fix_offsets[:, :, None] == fix_offsets[:, None, :]
    last = jnp.max(jnp.where(eq, f_ids[None, None, :], -1), axis=2)
    win_words = jnp.take_along_axis(fix_words, last, axis=1)

    offs2 = fix_offsets.reshape(NW, NFIX)
    words3 = win_words.reshape(NW, F, 128)

    return _embed_with_fixes(ids4, table, offs2, words3, word_embeddings)
